# Initial kernel scaffold; baseline (speedup 1.0000x reference)
import jax
import jax.numpy as jnp
from jax.experimental import pallas as pl


def kernel(x, edge_index, batch_index, a, Wi, bi, W1c, b1c, W2c, b2c, W3c, b3c, ln1_g, ln1_b, ln2_g, ln2_b, ln4_g, ln4_b, ln5_g, ln5_b, afc1_W, afc1_b, afc2_W, afc2_b, afc3_W, afc3_b, fc1_W, fc1_b, fc2_W, fc2_b):
    raise NotImplementedError("write your pallas kernel here")


# speedup vs baseline: 21.6132x; 21.6132x over previous
import dataclasses
import functools

import jax
import jax.numpy as jnp
from jax import lax
from jax.experimental import pallas as pl
from jax.experimental.pallas import tpu as pltpu
from jax.experimental.pallas import tpu_sc as plsc

N = 10000
E = 320000
G = 64
D = 128
NC = 2
NS = 16
NW = NC * NS
CE = E // NW
K = 125
NB = CE // K
PB = 16
NPH = NB // PB
ZTILE = 640
ZR = 80
ZCH = 8
DEG_L = 16
SEG_PER = G // NW
BCH = 64

_mesh = plsc.VectorSubcoreMesh(core_axis_name="c", subcore_axis_name="s")

_cp_no_layout = pltpu.CompilerParams()
if "needs_layout_passes" in pltpu.CompilerParams.__dataclass_fields__:
    _cp_no_layout = dataclasses.replace(_cp_no_layout, needs_layout_passes=False)

NEG_INF = float("-inf")


@functools.partial(
    pl.kernel,
    out_type=jax.ShapeDtypeStruct((NC, N, DEG_L), jnp.float32),
    mesh=_mesh,
    scratch_types=[
        pltpu.VMEM((NB, K), jnp.int32),
        pltpu.VMEM((K, DEG_L), jnp.float32),
        pltpu.VMEM((ZR, DEG_L), jnp.float32),
        pltpu.VMEM_SHARED((N, DEG_L), jnp.float32),
        pltpu.SemaphoreType.DMA,
    ],
)
def _sc_degree(dst_hbm, deg_out, didx, ones, zbuf, dacc, sem):
    c = lax.axis_index("c")
    s = lax.axis_index("s")
    wid = c * NS + s
    pltpu.async_copy(dst_hbm.at[wid], didx, sem).wait()

    @pl.loop(0, K)
    def _(r):
        ones[r, :] = jnp.ones((DEG_L,), jnp.float32)

    @pl.loop(0, ZR)
    def _(r):
        zbuf[r, :] = jnp.zeros((DEG_L,), jnp.float32)

    @pl.loop(0, ZCH)
    def _(i):
        off = s * ZTILE + i * ZR

        @pl.when(off + ZR <= N)
        def _():
            pltpu.sync_copy(zbuf, dacc.at[pl.ds(off, ZR)])

    plsc.subcore_barrier()

    @pl.loop(0, NB)
    def _(j):
        pltpu.sync_copy(ones, dacc.at[didx.at[j]], add=True)

    plsc.subcore_barrier()

    @pl.loop(0, ZCH)
    def _(i):
        off = s * ZTILE + i * ZR

        @pl.when(off + ZR <= N)
        def _():
            pltpu.sync_copy(dacc.at[pl.ds(off, ZR)],
                            deg_out.at[c, pl.ds(off, ZR)])


@functools.partial(
    pl.kernel,
    out_type=jax.ShapeDtypeStruct((NC, N, D), jnp.float32),
    mesh=_mesh,
    scratch_types=[
        pltpu.VMEM((PB, K), jnp.int32),
        pltpu.VMEM((PB, K), jnp.int32),
        pltpu.VMEM((2, K, D), jnp.float32),
        pltpu.VMEM_SHARED((N, D), jnp.float32),
        pltpu.SemaphoreType.DMA,
        pltpu.SemaphoreType.DMA,
        pltpu.SemaphoreType.DMA,
        pltpu.SemaphoreType.DMA,
    ],
)
def _sc_aggregate(tab_hbm, src_hbm, dst_hbm, out_hbm,
                  sidx, didx, stag, acc, g0, g1, s0, s1):
    c = lax.axis_index("c")
    s = lax.axis_index("s")
    wid = c * NS + s

    @pl.loop(0, ZR)
    def _(r):
        @pl.loop(0, D // 16)
        def _(v):
            stag[0, r, pl.ds(v * 16, 16)] = jnp.zeros((16,), jnp.float32)

    @pl.loop(0, ZCH)
    def _(i):
        off = s * ZTILE + i * ZR

        @pl.when(off + ZR <= N)
        def _():
            pltpu.sync_copy(stag.at[0, pl.ds(0, ZR)], acc.at[pl.ds(off, ZR)])

    plsc.subcore_barrier()

    @pl.loop(0, NPH)
    def _(p):
        pltpu.sync_copy(src_hbm.at[wid, pl.ds(p * PB, PB)], sidx)
        pltpu.sync_copy(dst_hbm.at[wid, pl.ds(p * PB, PB)], didx)
        pltpu.async_copy(tab_hbm.at[sidx.at[0]], stag.at[0], g0)

        @pl.loop(0, PB, step=2)
        def _(j):
            pltpu.async_copy(tab_hbm.at[sidx.at[j + 1]], stag.at[1], g1)
            pltpu.make_async_copy(tab_hbm.at[sidx.at[j]], stag.at[0], g0).wait()
            pltpu.async_copy(stag.at[0], acc.at[didx.at[j]], s0, add=True)
            pltpu.make_async_copy(stag.at[0], acc.at[didx.at[j]], s0).wait()

            @pl.when(j + 2 < PB)
            def _():
                pltpu.async_copy(tab_hbm.at[sidx.at[j + 2]], stag.at[0], g0)

            pltpu.make_async_copy(tab_hbm.at[sidx.at[j + 1]], stag.at[1], g1).wait()
            pltpu.async_copy(stag.at[1], acc.at[didx.at[j + 1]], s1, add=True)
            pltpu.make_async_copy(stag.at[1], acc.at[didx.at[j + 1]], s1).wait()

    plsc.subcore_barrier()

    @pl.loop(0, ZCH)
    def _(i):
        off = s * ZTILE + i * ZR

        @pl.when(off + ZR <= N)
        def _():
            pltpu.sync_copy(acc.at[pl.ds(off, ZR)],
                            out_hbm.at[c, pl.ds(off, ZR)])


@functools.partial(
    pl.kernel,
    out_type=jax.ShapeDtypeStruct((NW, SEG_PER, D), jnp.float32),
    mesh=_mesh,
    scratch_types=[
        pltpu.VMEM((N // 16, 16), jnp.int32),
        pltpu.VMEM((BCH, D), jnp.float32),
        pltpu.VMEM((SEG_PER, D), jnp.float32),
        pltpu.SemaphoreType.DMA,
    ],
    compiler_params=_cp_no_layout,
)
def _sc_pool(h_hbm, b_hbm, out_hbm, bidx, chunk, res, sem):
    c = lax.axis_index("c")
    s = lax.axis_index("s")
    wid = c * NS + s
    g0 = wid * SEG_PER
    pltpu.async_copy(b_hbm, bidx, sem).wait()

    zero = jnp.zeros((16,), jnp.int32)

    def count_body(r, carry):
        v = bidx[r]
        return tuple(
            carry[k] + plsc.all_reduce_population_count(v < (g0 + k))
            for k in range(SEG_PER + 1))

    counts = lax.fori_loop(0, N // 16, count_body, (zero,) * (SEG_PER + 1))
    starts = [jnp.max(counts[k]) for k in range(SEG_PER + 1)]

    neg = jnp.full((16,), NEG_INF, jnp.float32)
    for seg in range(SEG_PER):
        sstart = starts[seg]
        send = starts[seg + 1]
        astart = (sstart // 8) * 8
        nch = (send - astart + (BCH - 1)) // BCH

        def chunk_body(k, acc, sstart=sstart, send=send, astart=astart):
            cstart = astart + k * BCH
            lstart = jnp.minimum(cstart, N - BCH)
            pltpu.sync_copy(h_hbm.at[pl.ds(lstart, BCH)], chunk)

            def row_body(r, acc2):
                grow = lstart + r
                valid = (grow >= sstart) & (grow < send)
                out = []
                for v in range(D // 16):
                    x = chunk[r, pl.ds(v * 16, 16)]
                    out.append(jnp.maximum(acc2[v], jnp.where(valid, x, neg)))
                return tuple(out)

            return lax.fori_loop(0, BCH, row_body, acc)

        acc = lax.fori_loop(0, nch, chunk_body, (neg,) * (D // 16))
        for v in range(D // 16):
            res[seg, pl.ds(v * 16, 16)] = acc[v]

    pltpu.sync_copy(res, out_hbm.at[wid])


def _lrelu(x):
    return jnp.where(x >= 0, x, 0.01 * x)


def _tc_pre_body(deg_ref, x_ref, wi_ref, dinv_ref, hw0_ref, h0p_ref):
    deg = deg_ref[0, :, 0:1] + deg_ref[1, :, 0:1] + 1.0
    dinv = lax.rsqrt(deg)
    dinv_ref[...] = dinv
    x = x_ref[...]
    wi = wi_ref[...]
    hw0 = (x[:, 0:1] * wi[0:1, :] + x[:, 1:2] * wi[1:2, :]
           + x[:, 2:3] * wi[2:3, :])
    hw0_ref[...] = hw0
    h0p_ref[...] = hw0 * dinv


def _tc_pre(degp, x, wi):
    return pl.pallas_call(
        _tc_pre_body,
        out_shape=[
            jax.ShapeDtypeStruct((N, 1), jnp.float32),
            jax.ShapeDtypeStruct((N, D), jnp.float32),
            jax.ShapeDtypeStruct((N, D), jnp.float32),
        ],
    )(degp, x, wi)


def _tc_mid_first_body(agg_ref, hw_ref, dinv_ref, b_ref, wn_ref,
                       h_ref, hwn_ref, hpn_ref):
    dinv = dinv_ref[...]
    h = (dinv * (agg_ref[0] + agg_ref[1])
         + (dinv * dinv) * hw_ref[...] + b_ref[...])
    h_ref[...] = h
    hwn = jnp.dot(h, wn_ref[...], preferred_element_type=jnp.float32)
    hwn_ref[...] = hwn
    hpn_ref[...] = hwn * dinv


def _tc_mid_body(agg_ref, hw_ref, dinv_ref, b_ref, wn_ref, hprev_ref,
                 h_ref, hwn_ref, hpn_ref):
    dinv = dinv_ref[...]
    conv = (dinv * (agg_ref[0] + agg_ref[1])
            + (dinv * dinv) * hw_ref[...] + b_ref[...])
    h = _lrelu(conv) + hprev_ref[...]
    h_ref[...] = h
    hwn = jnp.dot(h, wn_ref[...], preferred_element_type=jnp.float32)
    hwn_ref[...] = hwn
    hpn_ref[...] = hwn * dinv


def _tc_mid(agg, hw, dinv, b, wn, hprev, first):
    out_shape = [
        jax.ShapeDtypeStruct((N, D), jnp.float32),
        jax.ShapeDtypeStruct((N, D), jnp.float32),
        jax.ShapeDtypeStruct((N, D), jnp.float32),
    ]
    if first:
        return pl.pallas_call(_tc_mid_first_body, out_shape=out_shape)(
            agg, hw, dinv, b, wn)
    return pl.pallas_call(_tc_mid_body, out_shape=out_shape)(
        agg, hw, dinv, b, wn, hprev)


def _tc_last_body(agg_ref, hw_ref, dinv_ref, b_ref, hprev_ref, h_ref):
    dinv = dinv_ref[...]
    conv = (dinv * (agg_ref[0] + agg_ref[1])
            + (dinv * dinv) * hw_ref[...] + b_ref[...])
    h_ref[...] = _lrelu(conv) + hprev_ref[...]


def _tc_last(agg, hw, dinv, b, hprev):
    return pl.pallas_call(
        _tc_last_body,
        out_shape=jax.ShapeDtypeStruct((N, D), jnp.float32),
    )(agg, hw, dinv, b, hprev)


def _ln(x, g, b):
    mu = jnp.mean(x, axis=-1, keepdims=True)
    var = jnp.mean((x - mu) ** 2, axis=-1, keepdims=True)
    return (x - mu) / jnp.sqrt(var + 1e-5) * g + b


def _tc_head_body(pooled_ref, a_ref, afc1t_ref, afc1b_ref, afc2t_ref,
                  afc2b_ref, afc3t_ref, afc3b_ref, fc1t_ref, fc1b_ref,
                  fc2t_ref, fc2b_ref, ln1g_ref, ln1b_ref, ln2g_ref, ln2b_ref,
                  ln4g_ref, ln4b_ref, ln5g_ref, ln5b_ref, out_ref):
    dot = functools.partial(jnp.dot, preferred_element_type=jnp.float32)
    v = _ln(pooled_ref[...], ln2g_ref[...], ln2b_ref[...])
    aa = _lrelu(dot(a_ref[...], afc1t_ref[...]) + afc1b_ref[...])
    aa = _ln(aa, ln5g_ref[...], ln5b_ref[...])
    aa = _lrelu(dot(aa, afc2t_ref[...]) + afc2b_ref[...])
    aa = _lrelu(dot(aa, afc3t_ref[...]) + afc3b_ref[...])
    aa = _ln(aa, ln1g_ref[...], ln1b_ref[...])
    z = v * aa
    z = _lrelu(dot(z, fc1t_ref[...]) + fc1b_ref[...])
    z = _ln(z, ln4g_ref[...], ln4b_ref[...])
    out_ref[...] = dot(z, fc2t_ref[...]) + fc2b_ref[...]


def _tc_head(*args):
    return pl.pallas_call(
        _tc_head_body,
        out_shape=jax.ShapeDtypeStruct((G, 1), jnp.float32),
    )(*args)


def kernel(x, edge_index, batch_index, a, Wi, bi, W1c, b1c, W2c, b2c, W3c,
           b3c, ln1_g, ln1_b, ln2_g, ln2_b, ln4_g, ln4_b, ln5_g, ln5_b,
           afc1_W, afc1_b, afc2_W, afc2_b, afc3_W, afc3_b,
           fc1_W, fc1_b, fc2_W, fc2_b):
    src3 = edge_index[0].reshape(NW, NB, K)
    dst3 = edge_index[1].reshape(NW, NB, K)
    b2d = batch_index.reshape(N // 16, 16)
    row = lambda t: t.reshape(1, -1)

    degp = _sc_degree(dst3)
    dinv, hw0, h0p = _tc_pre(degp, x, Wi)

    agg = _sc_aggregate(h0p, src3, dst3)
    h0, hw1, h1p = _tc_mid(agg, hw0, dinv, row(bi), W1c, None, True)
    agg = _sc_aggregate(h1p, src3, dst3)
    h1, hw2, h2p = _tc_mid(agg, hw1, dinv, row(b1c), W2c, h0, False)
    agg = _sc_aggregate(h2p, src3, dst3)
    h2, hw3, h3p = _tc_mid(agg, hw2, dinv, row(b2c), W3c, h1, False)
    agg = _sc_aggregate(h3p, src3, dst3)
    h3 = _tc_last(agg, hw3, dinv, row(b3c), h2)

    pooled = _sc_pool(h3, b2d).reshape(G, D)

    z = _tc_head(pooled, a, afc1_W.T, row(afc1_b), afc2_W.T, row(afc2_b),
                 afc3_W.T, row(afc3_b), fc1_W.T, row(fc1_b), fc2_W.T,
                 row(fc2_b), row(ln1_g), row(ln1_b), row(ln2_g), row(ln2_b),
                 row(ln4_g), row(ln4_b), row(ln5_g), row(ln5_b))
    return z.reshape(G)


# speedup vs baseline: 22.5145x; 1.0417x over previous
import dataclasses
import functools

import jax
import jax.numpy as jnp
from jax import lax
from jax.experimental import pallas as pl
from jax.experimental.pallas import tpu as pltpu
from jax.experimental.pallas import tpu_sc as plsc

N = 10000
E = 320000
G = 64
D = 128
NC = 2
NS = 16
NW = NC * NS
CE = E // NW
K = 125
NB = CE // K
PB = 16
NPH = NB // PB
ZTILE = 640
ZR = 80
ZCH = 8
DEG_L = 16
SEG_PER = G // NW
BCH = 64

def _mesh():
    return plsc.VectorSubcoreMesh(core_axis_name="c", subcore_axis_name="s",
                                  num_cores=NC, num_subcores=NS)

_cp_no_layout = pltpu.CompilerParams()
if "needs_layout_passes" in pltpu.CompilerParams.__dataclass_fields__:
    _cp_no_layout = dataclasses.replace(_cp_no_layout, needs_layout_passes=False)

NEG_INF = float("-inf")


def _sc_degree_body(dst_hbm, deg_out, didx, ones, zbuf, dacc, sem):
    c = lax.axis_index("c")
    s = lax.axis_index("s")
    wid = c * NS + s
    pltpu.async_copy(dst_hbm.at[wid], didx, sem).wait()

    @pl.loop(0, K)
    def _(r):
        ones[r, :] = jnp.ones((DEG_L,), jnp.float32)

    @pl.loop(0, ZR)
    def _(r):
        zbuf[r, :] = jnp.zeros((DEG_L,), jnp.float32)

    @pl.loop(0, ZCH)
    def _(i):
        off = s * ZTILE + i * ZR

        @pl.when(off + ZR <= N)
        def _():
            pltpu.sync_copy(zbuf, dacc.at[pl.ds(off, ZR)])

    plsc.subcore_barrier()

    @pl.loop(0, NB)
    def _(j):
        pltpu.sync_copy(ones, dacc.at[didx.at[j]], add=True)

    plsc.subcore_barrier()

    @pl.loop(0, ZCH)
    def _(i):
        off = s * ZTILE + i * ZR

        @pl.when(off + ZR <= N)
        def _():
            pltpu.sync_copy(dacc.at[pl.ds(off, ZR)],
                            deg_out.at[c, pl.ds(off, ZR)])


@functools.cache
def _sc_degree_call():
    return pl.kernel(
        _sc_degree_body,
        out_type=jax.ShapeDtypeStruct((NC, N, DEG_L), jnp.float32),
        mesh=_mesh(),
        scratch_types=[
            pltpu.VMEM((NB, K), jnp.int32),
            pltpu.VMEM((K, DEG_L), jnp.float32),
            pltpu.VMEM((ZR, DEG_L), jnp.float32),
            pltpu.VMEM_SHARED((N, DEG_L), jnp.float32),
            pltpu.SemaphoreType.DMA,
        ],
    )


def _sc_degree(dst3):
    return _sc_degree_call()(dst3)


def _sc_aggregate_body(tab_hbm, src_hbm, dst_hbm, out_hbm,
                       sidx, didx, stag, acc, g0, g1, s0, s1):
    c = lax.axis_index("c")
    s = lax.axis_index("s")
    wid = c * NS + s

    @pl.loop(0, ZR)
    def _(r):
        @pl.loop(0, D // 16)
        def _(v):
            stag[0, r, pl.ds(v * 16, 16)] = jnp.zeros((16,), jnp.float32)

    @pl.loop(0, ZCH)
    def _(i):
        off = s * ZTILE + i * ZR

        @pl.when(off + ZR <= N)
        def _():
            pltpu.sync_copy(stag.at[0, pl.ds(0, ZR)], acc.at[pl.ds(off, ZR)])

    plsc.subcore_barrier()

    @pl.loop(0, NPH)
    def _(p):
        pltpu.sync_copy(src_hbm.at[wid, pl.ds(p * PB, PB)], sidx)
        pltpu.sync_copy(dst_hbm.at[wid, pl.ds(p * PB, PB)], didx)
        pltpu.async_copy(tab_hbm.at[sidx.at[0]], stag.at[0], g0)

        @pl.loop(0, PB, step=2)
        def _(j):
            pltpu.async_copy(tab_hbm.at[sidx.at[j + 1]], stag.at[1], g1)
            pltpu.make_async_copy(tab_hbm.at[sidx.at[j]], stag.at[0], g0).wait()
            pltpu.async_copy(stag.at[0], acc.at[didx.at[j]], s0, add=True)
            pltpu.make_async_copy(stag.at[0], acc.at[didx.at[j]], s0).wait()

            @pl.when(j + 2 < PB)
            def _():
                pltpu.async_copy(tab_hbm.at[sidx.at[j + 2]], stag.at[0], g0)

            pltpu.make_async_copy(tab_hbm.at[sidx.at[j + 1]], stag.at[1], g1).wait()
            pltpu.async_copy(stag.at[1], acc.at[didx.at[j + 1]], s1, add=True)
            pltpu.make_async_copy(stag.at[1], acc.at[didx.at[j + 1]], s1).wait()

    plsc.subcore_barrier()

    @pl.loop(0, ZCH)
    def _(i):
        off = s * ZTILE + i * ZR

        @pl.when(off + ZR <= N)
        def _():
            pltpu.sync_copy(acc.at[pl.ds(off, ZR)],
                            out_hbm.at[c, pl.ds(off, ZR)])


@functools.cache
def _sc_aggregate_call():
    return pl.kernel(
        _sc_aggregate_body,
        out_type=jax.ShapeDtypeStruct((NC, N, D), jnp.float32),
        mesh=_mesh(),
        scratch_types=[
            pltpu.VMEM((PB, K), jnp.int32),
            pltpu.VMEM((PB, K), jnp.int32),
            pltpu.VMEM((2, K, D), jnp.float32),
            pltpu.VMEM_SHARED((N, D), jnp.float32),
            pltpu.SemaphoreType.DMA,
            pltpu.SemaphoreType.DMA,
            pltpu.SemaphoreType.DMA,
            pltpu.SemaphoreType.DMA,
        ],
    )


def _sc_aggregate(table, src3, dst3):
    return _sc_aggregate_call()(table, src3, dst3)


def _sc_pool_body(h_hbm, b_hbm, out_hbm, bidx, chunk, res, sem):
    c = lax.axis_index("c")
    s = lax.axis_index("s")
    wid = c * NS + s
    g0 = wid * SEG_PER
    pltpu.async_copy(b_hbm, bidx, sem).wait()

    zero = jnp.zeros((16,), jnp.int32)

    def count_body(r, carry):
        v = bidx[r]
        return tuple(
            carry[k] + plsc.all_reduce_population_count(v < (g0 + k))
            for k in range(SEG_PER + 1))

    counts = lax.fori_loop(0, N // 16, count_body, (zero,) * (SEG_PER + 1))
    starts = [jnp.max(counts[k]) for k in range(SEG_PER + 1)]

    neg = jnp.full((16,), NEG_INF, jnp.float32)
    for seg in range(SEG_PER):
        sstart = starts[seg]
        send = starts[seg + 1]
        astart = (sstart // 8) * 8
        nch = (send - astart + (BCH - 1)) // BCH

        def chunk_body(k, acc, sstart=sstart, send=send, astart=astart):
            cstart = astart + k * BCH
            lstart = jnp.minimum(cstart, N - BCH)
            pltpu.sync_copy(h_hbm.at[pl.ds(lstart, BCH)], chunk)

            def row_body(r, acc2):
                grow = lstart + r
                valid = (grow >= sstart) & (grow < send)
                out = []
                for v in range(D // 16):
                    x = chunk[r, pl.ds(v * 16, 16)]
                    out.append(jnp.maximum(acc2[v], jnp.where(valid, x, neg)))
                return tuple(out)

            return lax.fori_loop(0, BCH, row_body, acc)

        acc = lax.fori_loop(0, nch, chunk_body, (neg,) * (D // 16))
        for v in range(D // 16):
            res[seg, pl.ds(v * 16, 16)] = acc[v]

    pltpu.sync_copy(res, out_hbm.at[wid])


@functools.cache
def _sc_pool_call():
    return pl.kernel(
        _sc_pool_body,
        out_type=jax.ShapeDtypeStruct((NW, SEG_PER, D), jnp.float32),
        mesh=_mesh(),
        scratch_types=[
            pltpu.VMEM((N // 16, 16), jnp.int32),
            pltpu.VMEM((BCH, D), jnp.float32),
            pltpu.VMEM((SEG_PER, D), jnp.float32),
            pltpu.SemaphoreType.DMA,
        ],
        compiler_params=_cp_no_layout,
    )


def _sc_pool(h3, b2d):
    return _sc_pool_call()(h3, b2d)


def _lrelu(x):
    return jnp.where(x >= 0, x, 0.01 * x)


BLK = 1000
GRID = N // BLK


def _nd_spec():
    return pl.BlockSpec((BLK, D), lambda i: (i, 0))


def _tc_hw0_body(x_ref, wi_ref, hw0_ref):
    x = x_ref[...]
    wi = wi_ref[...]
    hw0_ref[...] = (x[:, 0:1] * wi[0:1, :] + x[:, 1:2] * wi[1:2, :]
                    + x[:, 2:3] * wi[2:3, :])


def _tc_hw0(x, wi):
    return pl.pallas_call(
        _tc_hw0_body,
        grid=(GRID,),
        in_specs=[pl.BlockSpec((BLK, 3), lambda i: (i, 0)),
                  pl.BlockSpec((3, D), lambda i: (0, 0))],
        out_specs=_nd_spec(),
        out_shape=jax.ShapeDtypeStruct((N, D), jnp.float32),
    )(x, wi)


def _tc_pre_body(deg_ref, hw0_ref, dinv_ref, h0p_ref):
    deg = deg_ref[0, :, 0:1] + deg_ref[1, :, 0:1] + 1.0
    dinv = lax.rsqrt(deg)
    dinv_ref[...] = dinv
    h0p_ref[...] = hw0_ref[...] * dinv


def _tc_pre(degp, hw0):
    return pl.pallas_call(
        _tc_pre_body,
        grid=(GRID,),
        in_specs=[pl.BlockSpec((NC, BLK, DEG_L), lambda i: (0, i, 0)),
                  _nd_spec()],
        out_specs=[pl.BlockSpec((BLK, 1), lambda i: (i, 0)), _nd_spec()],
        out_shape=[
            jax.ShapeDtypeStruct((N, 1), jnp.float32),
            jax.ShapeDtypeStruct((N, D), jnp.float32),
        ],
    )(degp, hw0)


def _tc_mid_first_body(agg_ref, hp_ref, dinv_ref, b_ref, wn_ref,
                       h_ref, hpn_ref):
    dinv = dinv_ref[...]
    h = dinv * (agg_ref[0] + agg_ref[1] + hp_ref[...]) + b_ref[...]
    h_ref[...] = h
    hpn_ref[...] = dinv * jnp.dot(h, wn_ref[...],
                                  preferred_element_type=jnp.float32)


def _tc_mid_body(agg_ref, hp_ref, dinv_ref, b_ref, wn_ref, hprev_ref,
                 h_ref, hpn_ref):
    dinv = dinv_ref[...]
    conv = dinv * (agg_ref[0] + agg_ref[1] + hp_ref[...]) + b_ref[...]
    h = _lrelu(conv) + hprev_ref[...]
    h_ref[...] = h
    hpn_ref[...] = dinv * jnp.dot(h, wn_ref[...],
                                  preferred_element_type=jnp.float32)


def _tc_mid(agg, hp, dinv, b, wn, hprev, first):
    in_specs = [
        pl.BlockSpec((NC, BLK, D), lambda i: (0, i, 0)),
        _nd_spec(),
        pl.BlockSpec((BLK, 1), lambda i: (i, 0)),
        pl.BlockSpec((1, D), lambda i: (0, 0)),
        pl.BlockSpec((D, D), lambda i: (0, 0)),
    ]
    out_shape = [
        jax.ShapeDtypeStruct((N, D), jnp.float32),
        jax.ShapeDtypeStruct((N, D), jnp.float32),
    ]
    out_specs = [_nd_spec(), _nd_spec()]
    if first:
        return pl.pallas_call(
            _tc_mid_first_body, grid=(GRID,), in_specs=in_specs,
            out_specs=out_specs, out_shape=out_shape)(agg, hp, dinv, b, wn)
    return pl.pallas_call(
        _tc_mid_body, grid=(GRID,), in_specs=in_specs + [_nd_spec()],
        out_specs=out_specs, out_shape=out_shape)(agg, hp, dinv, b, wn, hprev)


def _tc_last_body(agg_ref, hp_ref, dinv_ref, b_ref, hprev_ref, h_ref):
    dinv = dinv_ref[...]
    conv = dinv * (agg_ref[0] + agg_ref[1] + hp_ref[...]) + b_ref[...]
    h_ref[...] = _lrelu(conv) + hprev_ref[...]


def _tc_last(agg, hp, dinv, b, hprev):
    return pl.pallas_call(
        _tc_last_body,
        grid=(GRID,),
        in_specs=[
            pl.BlockSpec((NC, BLK, D), lambda i: (0, i, 0)),
            _nd_spec(),
            pl.BlockSpec((BLK, 1), lambda i: (i, 0)),
            pl.BlockSpec((1, D), lambda i: (0, 0)),
            _nd_spec(),
        ],
        out_specs=_nd_spec(),
        out_shape=jax.ShapeDtypeStruct((N, D), jnp.float32),
    )(agg, hp, dinv, b, hprev)


def _ln(x, g, b):
    mu = jnp.mean(x, axis=-1, keepdims=True)
    var = jnp.mean((x - mu) ** 2, axis=-1, keepdims=True)
    return (x - mu) / jnp.sqrt(var + 1e-5) * g + b


def _tc_head_body(pooled_ref, a_ref, afc1t_ref, afc1b_ref, afc2t_ref,
                  afc2b_ref, afc3t_ref, afc3b_ref, fc1t_ref, fc1b_ref,
                  fc2t_ref, fc2b_ref, ln1g_ref, ln1b_ref, ln2g_ref, ln2b_ref,
                  ln4g_ref, ln4b_ref, ln5g_ref, ln5b_ref, out_ref):
    dot = functools.partial(jnp.dot, preferred_element_type=jnp.float32)
    v = _ln(pooled_ref[...], ln2g_ref[...], ln2b_ref[...])
    aa = _lrelu(dot(a_ref[...], afc1t_ref[...]) + afc1b_ref[...])
    aa = _ln(aa, ln5g_ref[...], ln5b_ref[...])
    aa = _lrelu(dot(aa, afc2t_ref[...]) + afc2b_ref[...])
    aa = _lrelu(dot(aa, afc3t_ref[...]) + afc3b_ref[...])
    aa = _ln(aa, ln1g_ref[...], ln1b_ref[...])
    z = v * aa
    z = _lrelu(dot(z, fc1t_ref[...]) + fc1b_ref[...])
    z = _ln(z, ln4g_ref[...], ln4b_ref[...])
    out_ref[...] = dot(z, fc2t_ref[...]) + fc2b_ref[...]


def _tc_head(*args):
    return pl.pallas_call(
        _tc_head_body,
        out_shape=jax.ShapeDtypeStruct((G, 1), jnp.float32),
    )(*args)


def kernel(x, edge_index, batch_index, a, Wi, bi, W1c, b1c, W2c, b2c, W3c,
           b3c, ln1_g, ln1_b, ln2_g, ln2_b, ln4_g, ln4_b, ln5_g, ln5_b,
           afc1_W, afc1_b, afc2_W, afc2_b, afc3_W, afc3_b,
           fc1_W, fc1_b, fc2_W, fc2_b):
    src3 = edge_index[0].reshape(NW, NB, K)
    dst3 = edge_index[1].reshape(NW, NB, K)
    b2d = batch_index.reshape(N // 16, 16)
    row = lambda t: t.reshape(1, -1)

    degp = _sc_degree(dst3)
    hw0 = _tc_hw0(x, Wi)
    dinv, h0p = _tc_pre(degp, hw0)

    agg = _sc_aggregate(h0p, src3, dst3)
    h0, h1p = _tc_mid(agg, h0p, dinv, row(bi), W1c, None, True)
    agg = _sc_aggregate(h1p, src3, dst3)
    h1, h2p = _tc_mid(agg, h1p, dinv, row(b1c), W2c, h0, False)
    agg = _sc_aggregate(h2p, src3, dst3)
    h2, h3p = _tc_mid(agg, h2p, dinv, row(b2c), W3c, h1, False)
    agg = _sc_aggregate(h3p, src3, dst3)
    h3 = _tc_last(agg, h3p, dinv, row(b3c), h2)

    pooled = _sc_pool(h3, b2d).reshape(G, D)

    z = _tc_head(pooled, a, afc1_W.T, row(afc1_b), afc2_W.T, row(afc2_b),
                 afc3_W.T, row(afc3_b), fc1_W.T, row(fc1_b), fc2_W.T,
                 row(fc2_b), row(ln1_g), row(ln1_b), row(ln2_g), row(ln2_b),
                 row(ln4_g), row(ln4_b), row(ln5_g), row(ln5_b))
    return z.reshape(G)
